# Initial kernel scaffold; baseline (speedup 1.0000x reference)
import functools

import jax
import jax.numpy as jnp
from jax import lax
from jax.experimental import pallas as pl
from jax.experimental.pallas import tpu as pltpu
from jax.experimental.pallas import tpu_sc as plsc

N = 10000
E = 320000
D = 128

NW = 32
EPW = E // NW
CH = 80
NCH = EPW // CH
NPAD = 10240
SEG = NPAD // 16

ROWB = 1000

_mesh = plsc.VectorSubcoreMesh(core_axis_name="c", subcore_axis_name="s")


def _dense_in_body(x_ref, ws_ref, wd_ref, asv_ref, adv_ref,
                   h_ref, as_ref, ad_ref, m_ref, acc):
    i = pl.program_id(0)
    h = jnp.dot(x_ref[...], ws_ref[...], preferred_element_type=jnp.float32)
    h_ref[...] = h
    a_s = jnp.dot(h, asv_ref[...], preferred_element_type=jnp.float32)
    as_ref[...] = a_s
    v_d = jnp.dot(wd_ref[...], adv_ref[...], preferred_element_type=jnp.float32)
    a_d = jnp.dot(x_ref[...], v_d, preferred_element_type=jnp.float32)
    ad_ref[...] = a_d
    bs = jnp.max(a_s)
    bd = jnp.max(a_d)

    @pl.when(i == 0)
    def _():
        acc[0] = bs
        acc[1] = bd

    @pl.when(i > 0)
    def _():
        acc[0] = jnp.maximum(acc[0], bs)
        acc[1] = jnp.maximum(acc[1], bd)

    @pl.when(i == pl.num_programs(0) - 1)
    def _():
        m = acc[0] + acc[1]
        m = jnp.where(m >= 0.0, m, m * 0.2)
        m_ref[...] = jnp.full((8, 128), m, jnp.float32)


_dense_in = pl.pallas_call(
    _dense_in_body,
    grid=(N // ROWB,),
    in_specs=[
        pl.BlockSpec((ROWB, D), lambda i: (i, 0)),
        pl.BlockSpec((D, D), lambda i: (0, 0)),
        pl.BlockSpec((D, D), lambda i: (0, 0)),
        pl.BlockSpec((D, 1), lambda i: (0, 0)),
        pl.BlockSpec((D, 1), lambda i: (0, 0)),
    ],
    out_specs=[
        pl.BlockSpec((ROWB, D), lambda i: (i, 0)),
        pl.BlockSpec((ROWB, 1), lambda i: (i, 0)),
        pl.BlockSpec((ROWB, 1), lambda i: (i, 0)),
        pl.BlockSpec((8, 128), lambda i: (0, 0)),
    ],
    out_shape=[
        jax.ShapeDtypeStruct((N, D), jnp.float32),
        jax.ShapeDtypeStruct((N, 1), jnp.float32),
        jax.ShapeDtypeStruct((N, 1), jnp.float32),
        jax.ShapeDtypeStruct((8, 128), jnp.float32),
    ],
    scratch_shapes=[pltpu.SMEM((2,), jnp.float32)],
)


@functools.partial(
    pl.kernel,
    mesh=_mesh,
    out_type=[
        jax.ShapeDtypeStruct((NW, NCH, CH), jnp.float32),
        jax.ShapeDtypeStruct((2, NPAD), jnp.float32),
    ],
    scratch_types=[
        pltpu.VMEM((N,), jnp.float32),
        pltpu.VMEM((N,), jnp.float32),
        pltpu.VMEM((NCH, CH), jnp.int32),
        pltpu.VMEM((NCH, CH), jnp.int32),
        pltpu.VMEM((NCH, CH), jnp.float32),
        pltpu.VMEM((16,), jnp.float32),
        pltpu.VMEM((SEG,), jnp.float32),
        pltpu.VMEM_SHARED((NPAD,), jnp.float32),
    ],
)
def _edge_pass_a(a_src_hbm, a_dst_hbm, edge_hbm, m_hbm,
                 ex_hbm, dpart_hbm,
                 a_src_t, a_dst_t, src_t, dst_t, ex_t, m_t, z_t, denom_sh):
    c = lax.axis_index("c")
    s = lax.axis_index("s")
    wid = c * 16 + s

    pltpu.sync_copy(a_src_hbm, a_src_t)
    pltpu.sync_copy(a_dst_hbm, a_dst_t)
    pltpu.sync_copy(edge_hbm.at[0, wid], src_t)
    pltpu.sync_copy(edge_hbm.at[1, wid], dst_t)
    pltpu.sync_copy(m_hbm, m_t)

    def zinit(i, _):
        z_t[pl.ds(i * 16, 16)] = jnp.zeros((16,), jnp.float32)
        return 0
    lax.fori_loop(0, SEG // 16, zinit, 0)
    pltpu.sync_copy(z_t, denom_sh.at[pl.ds(s * SEG, SEG)])
    plsc.subcore_barrier()

    m_v = m_t[...]

    def chunk(j, _):
        for k in range(CH // 16):
            sl = pl.ds(k * 16, 16)
            sv = src_t[j, sl]
            dv = dst_t[j, sl]
            av = plsc.load_gather(a_src_t, [sv])
            bv = plsc.load_gather(a_dst_t, [dv])
            e = av + bv
            e = jnp.where(e >= 0.0, e, e * 0.2)
            ex_t[j, sl] = jnp.exp(e - m_v)
        pltpu.sync_copy(ex_t.at[j], denom_sh.at[dst_t.at[j]], add=True)
        return 0
    lax.fori_loop(0, NCH, chunk, 0)

    pltpu.sync_copy(ex_t, ex_hbm.at[wid])
    plsc.subcore_barrier()
    pltpu.sync_copy(denom_sh.at[pl.ds(s * SEG, SEG)],
                    dpart_hbm.at[c, pl.ds(s * SEG, SEG)])


@functools.partial(
    pl.kernel,
    mesh=_mesh,
    out_type=jax.ShapeDtypeStruct((2, NPAD, D), jnp.float32),
    scratch_types=[
        pltpu.VMEM((NPAD,), jnp.float32),
        pltpu.VMEM((NPAD,), jnp.float32),
        pltpu.VMEM((NCH, CH), jnp.int32),
        pltpu.VMEM((NCH, CH), jnp.int32),
        pltpu.VMEM((NCH, CH), jnp.float32),
        pltpu.VMEM((CH, D), jnp.float32),
        pltpu.VMEM((CH, D), jnp.float32),
        pltpu.VMEM_SHARED((NPAD, D), jnp.float32),
        pltpu.SemaphoreType.DMA,
    ],
)
def _edge_pass_b(h_hbm, edge_hbm, ex_hbm, dpart_hbm,
                 out_hbm,
                 den_t, dp1_t, src_t, dst_t, al_t, rows_t, z_t, out_sh, sem):
    c = lax.axis_index("c")
    s = lax.axis_index("s")
    wid = c * 16 + s

    pltpu.sync_copy(dpart_hbm.at[0], den_t)
    pltpu.sync_copy(dpart_hbm.at[1], dp1_t)
    pltpu.sync_copy(edge_hbm.at[0, wid], src_t)
    pltpu.sync_copy(edge_hbm.at[1, wid], dst_t)
    pltpu.sync_copy(ex_hbm.at[wid], al_t)

    def dcomb(i, _):
        sl = pl.ds(i * 16, 16)
        den_t[sl] = den_t[sl] + dp1_t[sl]
        return 0
    lax.fori_loop(0, NPAD // 16, dcomb, 0)

    def zrow(i, _):
        for q in range(D // 16):
            z_t[i, pl.ds(q * 16, 16)] = jnp.zeros((16,), jnp.float32)
        return 0
    lax.fori_loop(0, CH, zrow, 0)

    def zseg(b, _):
        pltpu.sync_copy(z_t, out_sh.at[pl.ds(s * SEG + b * CH, CH)])
        return 0
    lax.fori_loop(0, SEG // CH, zseg, 0)

    def alph(j, _):
        for k in range(CH // 16):
            sl = pl.ds(k * 16, 16)
            dv = dst_t[j, sl]
            den = plsc.load_gather(den_t, [dv])
            al_t[j, sl] = al_t[j, sl] / (den + 1e-16)
        return 0
    lax.fori_loop(0, NCH, alph, 0)

    plsc.subcore_barrier()

    def main(j, _):
        pltpu.async_copy(h_hbm.at[src_t.at[j]], rows_t, sem).wait()
        jv = jnp.broadcast_to(j, (16,)).astype(jnp.int32)

        def row(i, _2):
            iv = jnp.broadcast_to(i, (16,)).astype(jnp.int32)
            ab = plsc.load_gather(al_t, [jv, iv])
            for q in range(D // 16):
                sl = pl.ds(q * 16, 16)
                rows_t[i, sl] = rows_t[i, sl] * ab
            return 0
        lax.fori_loop(0, CH, row, 0)
        pltpu.sync_copy(rows_t, out_sh.at[dst_t.at[j]], add=True)
        return 0
    lax.fori_loop(0, NCH, main, 0)

    plsc.subcore_barrier()

    def wb(b, _):
        r0 = s * SEG + b * CH
        pltpu.sync_copy(out_sh.at[pl.ds(r0, CH)], out_hbm.at[c, pl.ds(r0, CH)])
        return 0
    lax.fori_loop(0, SEG // CH, wb, 0)


def _dense_out_body(o0_ref, o1_ref, bc_ref, wl_ref, bl_ref, y_ref):
    o = o0_ref[0] + o1_ref[0] + bc_ref[...]
    o = jnp.maximum(o, 0.0)
    y_ref[...] = jnp.dot(o, wl_ref[...],
                         preferred_element_type=jnp.float32) + bl_ref[...]


_dense_out = pl.pallas_call(
    _dense_out_body,
    grid=(N // ROWB,),
    in_specs=[
        pl.BlockSpec((1, ROWB, D), lambda i: (0, i, 0)),
        pl.BlockSpec((1, ROWB, D), lambda i: (1, i, 0)),
        pl.BlockSpec((1, D), lambda i: (0, 0)),
        pl.BlockSpec((D, D), lambda i: (0, 0)),
        pl.BlockSpec((1, D), lambda i: (0, 0)),
    ],
    out_specs=pl.BlockSpec((ROWB, D), lambda i: (i, 0)),
    out_shape=jax.ShapeDtypeStruct((N, D), jnp.float32),
)


def kernel(x, edge_index, W_src, W_dst, att_src, att_dst, b_conv, W_lin, b_lin):
    h_src, a_src, a_dst, m8 = _dense_in(
        x, W_src, W_dst, att_src.reshape(D, 1), att_dst.reshape(D, 1))
    edge_r = edge_index.reshape(2, NW, NCH, CH)
    m16 = jnp.broadcast_to(m8.reshape(-1)[:1], (16,))
    ex, dpart = _edge_pass_a(
        a_src.reshape(N), a_dst.reshape(N), edge_r, m16)
    outp = _edge_pass_b(h_src, edge_r, ex, dpart)
    y = _dense_out(outp, outp, b_conv.reshape(1, D), W_lin, b_lin.reshape(1, D))
    return y


# speedup vs baseline: 20.2261x; 20.2261x over previous
import functools

import jax
import jax.numpy as jnp
from jax import lax
from jax.experimental import pallas as pl
from jax.experimental.pallas import tpu as pltpu
from jax.experimental.pallas import tpu_sc as plsc

N = 10000
E = 320000
D = 128

NW = 32
EPW = E // NW
CH = 80
NCH = EPW // CH
NPAD = 10240
SEG = NPAD // 16

ROWB = 1000

_mesh = plsc.VectorSubcoreMesh(core_axis_name="c", subcore_axis_name="s")
_sc_params = pltpu.CompilerParams(needs_layout_passes=False,
                                  use_tc_tiling_on_sc=False)


def _dense_in_body(x_ref, ws_ref, wd_ref, asv_ref, adv_ref,
                   h_ref, as_ref, ad_ref, m_ref, acc):
    i = pl.program_id(0)
    h = jnp.dot(x_ref[...], ws_ref[...], preferred_element_type=jnp.float32)
    h_ref[0] = h[:, :64]
    h_ref[1] = h[:, 64:]
    a_s = jnp.dot(h, asv_ref[...], preferred_element_type=jnp.float32)
    as_ref[...] = a_s
    v_d = jnp.dot(wd_ref[...], adv_ref[...], preferred_element_type=jnp.float32)
    a_d = jnp.dot(x_ref[...], v_d, preferred_element_type=jnp.float32)
    ad_ref[...] = a_d
    bs = jnp.max(a_s)
    bd = jnp.max(a_d)

    @pl.when(i == 0)
    def _():
        acc[0] = bs
        acc[1] = bd

    @pl.when(i > 0)
    def _():
        acc[0] = jnp.maximum(acc[0], bs)
        acc[1] = jnp.maximum(acc[1], bd)

    @pl.when(i == pl.num_programs(0) - 1)
    def _():
        m = acc[0] + acc[1]
        m = jnp.where(m >= 0.0, m, m * 0.2)
        m_ref[...] = jnp.full((8, 128), m, jnp.float32)


_dense_in = pl.pallas_call(
    _dense_in_body,
    grid=(N // ROWB,),
    in_specs=[
        pl.BlockSpec((ROWB, D), lambda i: (i, 0)),
        pl.BlockSpec((D, D), lambda i: (0, 0)),
        pl.BlockSpec((D, D), lambda i: (0, 0)),
        pl.BlockSpec((D, 1), lambda i: (0, 0)),
        pl.BlockSpec((D, 1), lambda i: (0, 0)),
    ],
    out_specs=[
        pl.BlockSpec((2, ROWB, D // 2), lambda i: (0, i, 0)),
        pl.BlockSpec((ROWB, 1), lambda i: (i, 0)),
        pl.BlockSpec((ROWB, 1), lambda i: (i, 0)),
        pl.BlockSpec((8, 128), lambda i: (0, 0)),
    ],
    out_shape=[
        jax.ShapeDtypeStruct((2, N, D // 2), jnp.float32),
        jax.ShapeDtypeStruct((N, 1), jnp.float32),
        jax.ShapeDtypeStruct((N, 1), jnp.float32),
        jax.ShapeDtypeStruct((8, 128), jnp.float32),
    ],
    scratch_shapes=[pltpu.SMEM((2,), jnp.float32)],
)


@functools.partial(
    pl.kernel,
    mesh=_mesh,
    out_type=[
        jax.ShapeDtypeStruct((NW, NCH, CH), jnp.float32),
        jax.ShapeDtypeStruct((2, NPAD), jnp.float32),
    ],
    scratch_types=[
        pltpu.VMEM((N,), jnp.float32),
        pltpu.VMEM((N,), jnp.float32),
        pltpu.VMEM((NCH, CH), jnp.int32),
        pltpu.VMEM((NCH, CH), jnp.int32),
        pltpu.VMEM((NCH, CH), jnp.float32),
        pltpu.VMEM((16,), jnp.float32),
        pltpu.VMEM((SEG,), jnp.float32),
        pltpu.VMEM_SHARED((NPAD,), jnp.float32),
    ],
    compiler_params=_sc_params,
)
def _edge_pass_a(a_src_hbm, a_dst_hbm, edge_hbm, m_hbm,
                 ex_hbm, dpart_hbm,
                 a_src_t, a_dst_t, src_t, dst_t, ex_t, m_t, z_t, denom_sh):
    c = lax.axis_index("c")
    s = lax.axis_index("s")
    wid = c * 16 + s

    pltpu.sync_copy(a_src_hbm, a_src_t)
    pltpu.sync_copy(a_dst_hbm, a_dst_t)
    pltpu.sync_copy(edge_hbm.at[0, wid], src_t)
    pltpu.sync_copy(edge_hbm.at[1, wid], dst_t)
    pltpu.sync_copy(m_hbm, m_t)

    def zinit(i, _):
        z_t[pl.ds(i * 16, 16)] = jnp.zeros((16,), jnp.float32)
        return 0
    lax.fori_loop(0, SEG // 16, zinit, 0)
    pltpu.sync_copy(z_t, denom_sh.at[pl.ds(s * SEG, SEG)])
    plsc.subcore_barrier()

    m_v = m_t[...]

    def chunk(j, _):
        for k in range(CH // 16):
            sl = pl.ds(k * 16, 16)
            sv = src_t[j, sl]
            dv = dst_t[j, sl]
            av = plsc.load_gather(a_src_t, [sv])
            bv = plsc.load_gather(a_dst_t, [dv])
            e = av + bv
            e = jnp.where(e >= 0.0, e, e * 0.2)
            ex_t[j, sl] = jnp.exp(e - m_v)
        pltpu.sync_copy(ex_t.at[j], denom_sh.at[dst_t.at[j]], add=True)
        return 0
    lax.fori_loop(0, NCH, chunk, 0)

    pltpu.sync_copy(ex_t, ex_hbm.at[wid])
    plsc.subcore_barrier()
    pltpu.sync_copy(denom_sh.at[pl.ds(s * SEG, SEG)],
                    dpart_hbm.at[c, pl.ds(s * SEG, SEG)])


DH = D // 2
NCH3 = (E // CH) // 16


@functools.partial(
    pl.kernel,
    mesh=_mesh,
    out_type=jax.ShapeDtypeStruct((2, NPAD, DH), jnp.float32),
    scratch_types=[
        pltpu.VMEM((NPAD,), jnp.float32),
        pltpu.VMEM((NPAD,), jnp.float32),
        pltpu.VMEM((NCH3, CH), jnp.int32),
        pltpu.VMEM((NCH3, CH), jnp.int32),
        pltpu.VMEM((NCH3, CH), jnp.float32),
        pltpu.VMEM((CH, DH), jnp.float32),
        pltpu.VMEM_SHARED((NPAD, DH), jnp.float32),
        pltpu.SemaphoreType.DMA,
    ],
    compiler_params=_sc_params,
)
def _edge_pass_b(h_hbm, edge_hbm, ex_hbm, dpart_hbm,
                 out_hbm,
                 den_t, dp1_t, src_t, dst_t, al_t, rows_t, out_sh, sem):
    c = lax.axis_index("c")
    s = lax.axis_index("s")

    pltpu.sync_copy(dpart_hbm.at[0], den_t)
    pltpu.sync_copy(dpart_hbm.at[1], dp1_t)
    pltpu.sync_copy(edge_hbm.at[0, s], src_t)
    pltpu.sync_copy(edge_hbm.at[1, s], dst_t)
    pltpu.sync_copy(ex_hbm.at[s], al_t)

    def dcomb(i, _):
        sl = pl.ds(i * 16, 16)
        den_t[sl] = den_t[sl] + dp1_t[sl]
        return 0
    lax.fori_loop(0, NPAD // 16, dcomb, 0)

    def zrow(i, _):
        for q in range(DH // 16):
            rows_t[i, pl.ds(q * 16, 16)] = jnp.zeros((16,), jnp.float32)
        return 0
    lax.fori_loop(0, CH, zrow, 0)

    def zseg(b, _):
        pltpu.sync_copy(rows_t, out_sh.at[pl.ds(s * SEG + b * CH, CH)])
        return 0
    lax.fori_loop(0, SEG // CH, zseg, 0)

    def alph(j, _):
        for k in range(CH // 16):
            sl = pl.ds(k * 16, 16)
            dv = dst_t[j, sl]
            den = plsc.load_gather(den_t, [dv])
            al_t[j, sl] = al_t[j, sl] / (den + 1e-16)
        return 0
    lax.fori_loop(0, NCH3, alph, 0)

    plsc.subcore_barrier()

    def main(j, _):
        pltpu.async_copy(h_hbm.at[c].at[src_t.at[j]], rows_t, sem).wait()
        jv = jnp.broadcast_to(j, (16,)).astype(jnp.int32)

        def row(i, _2):
            iv = jnp.broadcast_to(i, (16,)).astype(jnp.int32)
            ab = plsc.load_gather(al_t, [jv, iv])
            for q in range(DH // 16):
                sl = pl.ds(q * 16, 16)
                rows_t[i, sl] = rows_t[i, sl] * ab
            return 0
        lax.fori_loop(0, CH, row, 0)
        pltpu.sync_copy(rows_t, out_sh.at[dst_t.at[j]], add=True)
        return 0
    lax.fori_loop(0, NCH3, main, 0)

    plsc.subcore_barrier()

    def wb(b, _):
        r0 = s * SEG + b * CH
        pltpu.sync_copy(out_sh.at[pl.ds(r0, CH)], out_hbm.at[c, pl.ds(r0, CH)])
        return 0
    lax.fori_loop(0, SEG // CH, wb, 0)


def _dense_out_body(oa_ref, ob_ref, bc_ref, wl_ref, bl_ref, y_ref):
    oa = jnp.maximum(oa_ref[0] + bc_ref[:, :DH], 0.0)
    ob = jnp.maximum(ob_ref[0] + bc_ref[:, DH:], 0.0)
    y = jnp.dot(oa, wl_ref[:DH, :], preferred_element_type=jnp.float32)
    y = y + jnp.dot(ob, wl_ref[DH:, :], preferred_element_type=jnp.float32)
    y_ref[...] = y + bl_ref[...]


_dense_out = pl.pallas_call(
    _dense_out_body,
    grid=(N // ROWB,),
    in_specs=[
        pl.BlockSpec((1, ROWB, DH), lambda i: (0, i, 0)),
        pl.BlockSpec((1, ROWB, DH), lambda i: (1, i, 0)),
        pl.BlockSpec((1, D), lambda i: (0, 0)),
        pl.BlockSpec((D, D), lambda i: (0, 0)),
        pl.BlockSpec((1, D), lambda i: (0, 0)),
    ],
    out_specs=pl.BlockSpec((ROWB, D), lambda i: (i, 0)),
    out_shape=jax.ShapeDtypeStruct((N, D), jnp.float32),
)


def kernel(x, edge_index, W_src, W_dst, att_src, att_dst, b_conv, W_lin, b_lin):
    h_src, a_src, a_dst, m8 = _dense_in(
        x, W_src, W_dst, att_src.reshape(D, 1), att_dst.reshape(D, 1))
    edge_a = edge_index.reshape(2, NW, NCH, CH)
    edge_b = edge_index.reshape(2, 16, NCH3, CH)
    m16 = jnp.broadcast_to(m8.reshape(-1)[:1], (16,))
    ex, dpart = _edge_pass_a(
        a_src.reshape(N), a_dst.reshape(N), edge_a, m16)
    outp = _edge_pass_b(h_src, edge_b, ex.reshape(16, NCH3, CH), dpart)
    y = _dense_out(outp, outp, b_conv.reshape(1, D), W_lin,
                   b_lin.reshape(1, D))
    return y


# speedup vs baseline: 33.1394x; 1.6384x over previous
import functools

import jax
import jax.numpy as jnp
from jax import lax
from jax.experimental import pallas as pl
from jax.experimental.pallas import tpu as pltpu
from jax.experimental.pallas import tpu_sc as plsc

N = 10000
E = 320000
D = 128

NW = 32
EPW = E // NW
CH = 80
NCH = EPW // CH
NPAD = 10240
SEG = NPAD // 16

ROWB = 1000

_mesh = plsc.VectorSubcoreMesh(core_axis_name="c", subcore_axis_name="s")
_sc_params = pltpu.CompilerParams(needs_layout_passes=False,
                                  use_tc_tiling_on_sc=False)


def _dense_in_body(x_ref, ws_ref, wd_ref, asv_ref, adv_ref,
                   h_ref, as_ref, ad_ref, m_ref, acc):
    i = pl.program_id(0)
    h = jnp.dot(x_ref[...], ws_ref[...], preferred_element_type=jnp.float32)
    h_ref[0] = h[:, :64]
    h_ref[1] = h[:, 64:]
    a_s = jnp.dot(h, asv_ref[...], preferred_element_type=jnp.float32)
    as_ref[...] = a_s
    v_d = jnp.dot(wd_ref[...], adv_ref[...], preferred_element_type=jnp.float32)
    a_d = jnp.dot(x_ref[...], v_d, preferred_element_type=jnp.float32)
    ad_ref[...] = a_d
    bs = jnp.max(a_s)
    bd = jnp.max(a_d)

    @pl.when(i == 0)
    def _():
        acc[0] = bs
        acc[1] = bd

    @pl.when(i > 0)
    def _():
        acc[0] = jnp.maximum(acc[0], bs)
        acc[1] = jnp.maximum(acc[1], bd)

    @pl.when(i == pl.num_programs(0) - 1)
    def _():
        m = acc[0] + acc[1]
        m = jnp.where(m >= 0.0, m, m * 0.2)
        m_ref[...] = jnp.full((8, 128), m, jnp.float32)


_dense_in = pl.pallas_call(
    _dense_in_body,
    grid=(N // ROWB,),
    in_specs=[
        pl.BlockSpec((ROWB, D), lambda i: (i, 0)),
        pl.BlockSpec((D, D), lambda i: (0, 0)),
        pl.BlockSpec((D, D), lambda i: (0, 0)),
        pl.BlockSpec((D, 1), lambda i: (0, 0)),
        pl.BlockSpec((D, 1), lambda i: (0, 0)),
    ],
    out_specs=[
        pl.BlockSpec((2, ROWB, D // 2), lambda i: (0, i, 0)),
        pl.BlockSpec((ROWB, 1), lambda i: (i, 0)),
        pl.BlockSpec((ROWB, 1), lambda i: (i, 0)),
        pl.BlockSpec((8, 128), lambda i: (0, 0)),
    ],
    out_shape=[
        jax.ShapeDtypeStruct((2, N, D // 2), jnp.float32),
        jax.ShapeDtypeStruct((N, 1), jnp.float32),
        jax.ShapeDtypeStruct((N, 1), jnp.float32),
        jax.ShapeDtypeStruct((8, 128), jnp.float32),
    ],
    scratch_shapes=[pltpu.SMEM((2,), jnp.float32)],
)


@functools.partial(
    pl.kernel,
    mesh=_mesh,
    out_type=[
        jax.ShapeDtypeStruct((NW, NCH, CH), jnp.float32),
        jax.ShapeDtypeStruct((2, NPAD), jnp.float32),
    ],
    scratch_types=[
        pltpu.VMEM((N,), jnp.float32),
        pltpu.VMEM((N,), jnp.float32),
        pltpu.VMEM((NCH, CH), jnp.int32),
        pltpu.VMEM((NCH, CH), jnp.int32),
        pltpu.VMEM((NCH, CH), jnp.float32),
        pltpu.VMEM((16,), jnp.float32),
        pltpu.VMEM((SEG,), jnp.float32),
        pltpu.VMEM_SHARED((NPAD,), jnp.float32),
    ],
    compiler_params=_sc_params,
)
def _edge_pass_a(a_src_hbm, a_dst_hbm, edge_hbm, m_hbm,
                 ex_hbm, dpart_hbm,
                 a_src_t, a_dst_t, src_t, dst_t, ex_t, m_t, z_t, denom_sh):
    c = lax.axis_index("c")
    s = lax.axis_index("s")
    wid = c * 16 + s

    pltpu.sync_copy(a_src_hbm, a_src_t)
    pltpu.sync_copy(a_dst_hbm, a_dst_t)
    pltpu.sync_copy(edge_hbm.at[0, wid], src_t)
    pltpu.sync_copy(edge_hbm.at[1, wid], dst_t)
    pltpu.sync_copy(m_hbm, m_t)

    def zinit(i, _):
        z_t[pl.ds(i * 16, 16)] = jnp.zeros((16,), jnp.float32)
        return 0
    lax.fori_loop(0, SEG // 16, zinit, 0)
    pltpu.sync_copy(z_t, denom_sh.at[pl.ds(s * SEG, SEG)])
    plsc.subcore_barrier()

    m_v = m_t[...]

    def chunk(j, _):
        for k in range(CH // 16):
            sl = pl.ds(k * 16, 16)
            sv = src_t[j, sl]
            dv = dst_t[j, sl]
            av = plsc.load_gather(a_src_t, [sv])
            bv = plsc.load_gather(a_dst_t, [dv])
            e = av + bv
            e = jnp.where(e >= 0.0, e, e * 0.2)
            ex_t[j, sl] = jnp.exp(e - m_v)
        pltpu.sync_copy(ex_t.at[j], denom_sh.at[dst_t.at[j]], add=True)
        return 0
    lax.fori_loop(0, NCH, chunk, 0)

    pltpu.sync_copy(ex_t, ex_hbm.at[wid])
    plsc.subcore_barrier()
    pltpu.sync_copy(denom_sh.at[pl.ds(s * SEG, SEG)],
                    dpart_hbm.at[c, pl.ds(s * SEG, SEG)])


DH = D // 2
NCH3 = (E // CH) // 16


@functools.partial(
    pl.kernel,
    mesh=_mesh,
    out_type=jax.ShapeDtypeStruct((2, NPAD, DH), jnp.float32),
    scratch_types=[
        pltpu.VMEM((NCH3, CH), jnp.int32),
        pltpu.VMEM((NCH3, CH), jnp.int32),
        pltpu.VMEM((NCH3, CH), jnp.float32),
        pltpu.VMEM((CH, DH), jnp.float32),
        pltpu.VMEM((CH, DH), jnp.float32),
        pltpu.VMEM_SHARED((NPAD, DH), jnp.float32),
        pltpu.SemaphoreType.DMA,
        pltpu.SemaphoreType.DMA,
    ],
    compiler_params=_sc_params,
)
def _edge_pass_b(h_hbm, edge_hbm, ex_hbm,
                 out_hbm,
                 src_t, dst_t, al_t, rows_t, rows1_t, out_sh,
                 sem, sem1):
    c = lax.axis_index("c")
    s = lax.axis_index("s")

    pltpu.sync_copy(edge_hbm.at[0, s], src_t)
    pltpu.sync_copy(edge_hbm.at[1, s], dst_t)
    pltpu.sync_copy(ex_hbm.at[s], al_t)

    def zrow(i, _):
        for q in range(DH // 16):
            rows_t[i, pl.ds(q * 16, 16)] = jnp.zeros((16,), jnp.float32)
        return 0
    lax.fori_loop(0, CH, zrow, 0)

    def zseg(b, _):
        pltpu.sync_copy(rows_t, out_sh.at[pl.ds(s * SEG + b * CH, CH)])
        return 0
    lax.fori_loop(0, SEG // CH, zseg, 0)

    plsc.subcore_barrier()

    bufs = (rows_t, rows1_t)
    sems = (sem, sem1)

    def _proc(j, rt):
        jv = jnp.broadcast_to(j, (16,)).astype(jnp.int32)

        def row(i, _2):
            iv = jnp.broadcast_to(i, (16,)).astype(jnp.int32)
            ab = plsc.load_gather(al_t, [jv, iv])
            for q in range(DH // 16):
                sl = pl.ds(q * 16, 16)
                rt[i, sl] = rt[i, sl] * ab
            return 0
        lax.fori_loop(0, CH, row, 0)
        pltpu.sync_copy(rt, out_sh.at[dst_t.at[j]], add=True)

    for b in range(2):
        pltpu.async_copy(h_hbm.at[c].at[src_t.at[b]], bufs[b], sems[b])

    def main(i, _):
        for b in range(2):
            j = i * 2 + b
            pltpu.make_async_copy(
                h_hbm.at[c].at[src_t.at[j]], bufs[b], sems[b]).wait()
            _proc(j, bufs[b])
            pltpu.async_copy(
                h_hbm.at[c].at[src_t.at[j + 2]], bufs[b], sems[b])
        return 0
    lax.fori_loop(0, NCH3 // 2 - 1, main, 0)

    for b in range(2):
        j = NCH3 - 2 + b
        pltpu.make_async_copy(
            h_hbm.at[c].at[src_t.at[j]], bufs[b], sems[b]).wait()
        _proc(j, bufs[b])

    plsc.subcore_barrier()

    def wb(b, _):
        r0 = s * SEG + b * CH
        pltpu.sync_copy(out_sh.at[pl.ds(r0, CH)], out_hbm.at[c, pl.ds(r0, CH)])
        return 0
    lax.fori_loop(0, SEG // CH, wb, 0)


def _dense_out_body(oa_ref, ob_ref, d0_ref, d1_ref, bc_ref, wl_ref, bl_ref,
                    y_ref):
    inv = 1.0 / (d0_ref[0] + d1_ref[0] + 1e-16)
    oa = jnp.maximum(oa_ref[0] * inv + bc_ref[:, :DH], 0.0)
    ob = jnp.maximum(ob_ref[0] * inv + bc_ref[:, DH:], 0.0)
    y = jnp.dot(oa, wl_ref[:DH, :], preferred_element_type=jnp.float32)
    y = y + jnp.dot(ob, wl_ref[DH:, :], preferred_element_type=jnp.float32)
    y_ref[...] = y + bl_ref[...]


_dense_out = pl.pallas_call(
    _dense_out_body,
    grid=(N // ROWB,),
    in_specs=[
        pl.BlockSpec((1, ROWB, DH), lambda i: (0, i, 0)),
        pl.BlockSpec((1, ROWB, DH), lambda i: (1, i, 0)),
        pl.BlockSpec((1, ROWB, 1), lambda i: (0, i, 0)),
        pl.BlockSpec((1, ROWB, 1), lambda i: (1, i, 0)),
        pl.BlockSpec((1, D), lambda i: (0, 0)),
        pl.BlockSpec((D, D), lambda i: (0, 0)),
        pl.BlockSpec((1, D), lambda i: (0, 0)),
    ],
    out_specs=pl.BlockSpec((ROWB, D), lambda i: (i, 0)),
    out_shape=jax.ShapeDtypeStruct((N, D), jnp.float32),
)


def kernel(x, edge_index, W_src, W_dst, att_src, att_dst, b_conv, W_lin, b_lin):
    h_src, a_src, a_dst, m8 = _dense_in(
        x, W_src, W_dst, att_src.reshape(D, 1), att_dst.reshape(D, 1))
    edge_a = edge_index.reshape(2, NW, NCH, CH)
    edge_b = edge_index.reshape(2, 16, NCH3, CH)
    m16 = jnp.broadcast_to(m8.reshape(-1)[:1], (16,))
    ex, dpart = _edge_pass_a(
        a_src.reshape(N), a_dst.reshape(N), edge_a, m16)
    outp = _edge_pass_b(h_src, edge_b, ex.reshape(16, NCH3, CH))
    dp = dpart.reshape(2, NPAD, 1)
    y = _dense_out(outp, outp, dp, dp, b_conv.reshape(1, D), W_lin,
                   b_lin.reshape(1, D))
    return y
